# baseline (speedup 1.0000x reference)
import functools

import jax
import jax.numpy as jnp
from jax import lax
from jax.experimental import pallas as pl
from jax.experimental.pallas import tpu as pltpu
from jax.experimental.pallas import tpu_sc as plsc

_K = 8
_L = 16
_NC, _NS = 2, 16
_NW = _NC * _NS
_B, _S, _C = 64, 8192, 128
_SEG8 = _S // 8
_NM64 = _S // 64
_CAP1 = 48
_CAP2 = 16
_CAP3 = 16
_NEG = float("-inf")

_SORT8 = [
    (0, 1), (2, 3), (4, 5), (6, 7),
    (0, 2), (1, 3), (4, 6), (5, 7),
    (1, 2), (5, 6),
    (0, 4), (1, 5), (2, 6), (3, 7),
    (2, 4), (3, 5),
    (1, 2), (3, 4), (5, 6),
]


def _sort8_desc(v):
    v = list(v)
    for i, j in _SORT8:
        hi = jnp.maximum(v[i], v[j])
        lo = jnp.minimum(v[i], v[j])
        v[i], v[j] = hi, lo
    return v


def _insert(r8, x):
    out = []
    carry = x
    for k in range(_K):
        hi = jnp.maximum(r8[k], carry)
        carry = jnp.minimum(r8[k], carry)
        out.append(hi)
    return out


def _eighth_of_16(vals16):
    a = _sort8_desc(vals16[:8])
    b = _sort8_desc(vals16[8:])
    top8 = [jnp.maximum(a[k], b[7 - k]) for k in range(_K)]
    t = top8[0]
    for k in range(1, _K):
        t = jnp.minimum(t, top8[k])
    return t


def _seg8_body(x_ref, o_ref):
    x = x_ref[0]
    m = x.reshape(_SEG8, 8, _C)
    o_ref[0] = jnp.max(m, axis=1)


def _seg8_max(inputs):
    return pl.pallas_call(
        _seg8_body,
        grid=(_B,),
        in_specs=[pl.BlockSpec((1, _S, _C), lambda b: (b, 0, 0))],
        out_specs=pl.BlockSpec((1, _SEG8, _C), lambda b: (b, 0, 0)),
        out_shape=jax.ShapeDtypeStruct((_B, _SEG8, _C), jnp.float32),
    )(inputs)


def _sc_body(xf_hbm, s8_hbm, out_hbm,
             s8v, m64v, idb, vlb, id2b, id3b, ixb, gbuf, outb, sem):
    wid = lax.axis_index("s") * _NC + lax.axis_index("c")
    iota = lax.broadcasted_iota(jnp.int32, (_L,), 0)
    neg = jnp.full((_L,), _NEG, jnp.float32)
    zero = jnp.zeros((_L,), jnp.int32)

    def task_body(i, _):
        tau = wid * 16 + i
        b = tau // 8
        c0 = (tau % 8) * _L
        pltpu.sync_copy(s8_hbm.at[b, :, pl.ds(c0, _L)], s8v)

        def m64_body(j, accs):
            accs = list(accs)
            for u in range(16):
                jj = j * 16 + u
                m = jnp.maximum(
                    jnp.maximum(jnp.maximum(s8v[jj * 8], s8v[jj * 8 + 1]),
                                jnp.maximum(s8v[jj * 8 + 2], s8v[jj * 8 + 3])),
                    jnp.maximum(jnp.maximum(s8v[jj * 8 + 4], s8v[jj * 8 + 5]),
                                jnp.maximum(s8v[jj * 8 + 6], s8v[jj * 8 + 7])))
                m64v[jj] = m
                accs[u] = jnp.maximum(accs[u], m)
            return tuple(accs)

        accs = lax.fori_loop(0, _NM64 // 16, m64_body, (neg,) * 16)
        t1 = _eighth_of_16(list(accs))

        def c1_body(j, cnt):
            v = m64v[j]
            m = v >= t1
            ok = jnp.logical_and(m, cnt < _CAP1)
            idx = jnp.minimum(cnt, _CAP1 - 1) * _L + iota
            plsc.store_scatter(vlb, [idx], v, mask=ok)
            plsc.store_scatter(idb, [idx], zero + j, mask=ok)
            return cnt + m.astype(jnp.int32)

        cnt1 = lax.fori_loop(0, _NM64, c1_body, zero)

        def t64_body(s, r):
            v = vlb[pl.ds(pl.multiple_of(s * _L, _L), _L)]
            v = jnp.where(cnt1 > s, v, _NEG)
            return tuple(_insert(list(r), v))

        r = lax.fori_loop(0, _CAP1, t64_body, ((neg,) * _K))
        t64 = r[_K - 1]

        def c2_body(s, cnt):
            v = vlb[pl.ds(s * _L, _L)]
            sid = idb[pl.ds(s * _L, _L)]
            m = jnp.logical_and(v >= t64, cnt1 > s)
            ok = jnp.logical_and(m, cnt < _CAP2)
            idx = jnp.minimum(cnt, _CAP2 - 1) * _L + iota
            plsc.store_scatter(id2b, [idx], sid, mask=ok)
            return cnt + m.astype(jnp.int32)

        cnt2 = lax.fori_loop(0, _CAP1, c2_body, zero)

        def t8_body(s2, r):
            valid = cnt2 > s2
            sid = id2b[pl.ds(pl.multiple_of(s2 * _L, _L), _L)]
            row2 = jnp.where(valid, sid, 0) * 8
            r = list(r)
            for u in range(8):
                val = plsc.load_gather(s8v, [row2 + u, iota])
                r = _insert(r, jnp.where(valid, val, _NEG))
            return tuple(r)

        r = lax.fori_loop(0, _CAP2, t8_body, ((neg,) * _K))
        t8 = r[_K - 1]

        def c3_body(s2, cnt):
            valid = cnt2 > s2
            sid = id2b[pl.ds(pl.multiple_of(s2 * _L, _L), _L)]
            sid = jnp.where(valid, sid, 0)
            row2 = sid * 8
            for u in range(8):
                val = plsc.load_gather(s8v, [row2 + u, iota])
                m = jnp.logical_and(val >= t8, valid)
                ok = jnp.logical_and(m, cnt < _CAP3)
                idx = jnp.minimum(cnt, _CAP3 - 1) * _L + iota
                plsc.store_scatter(id3b, [idx], sid * 8 + u, mask=ok)
                cnt = cnt + m.astype(jnp.int32)
            return cnt

        cnt3 = lax.fori_loop(0, _CAP2, c3_body, zero)

        bvec = b * (_S * _C) + c0 + iota
        for s3 in range(_CAP3):
            sid3 = id3b[pl.ds(s3 * _L, _L)]
            sid3 = jnp.where(cnt3 > s3, sid3, 0)
            base3 = sid3 * (8 * _C) + bvec
            for u in range(8):
                ixb[s3, pl.ds(u * _L, _L)] = base3 + u * _C

        copies = [pltpu.async_copy(xf_hbm.at[ixb.at[s3]], gbuf.at[s3], sem)
                  for s3 in range(_CAP3)]
        for cp in copies:
            cp.wait()

        def fin_body(s3, r):
            valid = cnt3 > s3
            r = list(r)
            for u in range(8):
                v = gbuf[s3, pl.ds(u * _L, _L)]
                r = _insert(r, jnp.where(valid, v, _NEG))
            return tuple(r)

        r8 = list(lax.fori_loop(0, _CAP3, fin_body, ((neg,) * _K)))

        for k in range(_K):
            outb[k] = r8[k]
        pltpu.sync_copy(outb, out_hbm.at[b, :, pl.ds(c0, _L)])
        return 0

    lax.fori_loop(0, 512 // _NW, task_body, 0)


def kernel(inputs):
    s8 = _seg8_max(inputs)
    xf = inputs.reshape(-1)
    mesh = plsc.VectorSubcoreMesh(
        core_axis_name="c", subcore_axis_name="s",
        num_cores=_NC, num_subcores=_NS)
    fn = functools.partial(
        pl.kernel,
        out_type=jax.ShapeDtypeStruct((_B, _K, _C), jnp.float32),
        mesh=mesh,
        scratch_types=[
            pltpu.VMEM((_SEG8, _L), jnp.float32),
            pltpu.VMEM((_NM64, _L), jnp.float32),
            pltpu.VMEM((_CAP1 * _L,), jnp.int32),
            pltpu.VMEM((_CAP1 * _L,), jnp.float32),
            pltpu.VMEM((_CAP2 * _L,), jnp.int32),
            pltpu.VMEM((_CAP3 * _L,), jnp.int32),
            pltpu.VMEM((_CAP3, 8 * _L), jnp.int32),
            pltpu.VMEM((_CAP3, 8 * _L), jnp.float32),
            pltpu.VMEM((_K, _L), jnp.float32),
            pltpu.SemaphoreType.DMA,
        ],
        compiler_params=pltpu.CompilerParams(
            use_tc_tiling_on_sc=False, needs_layout_passes=False),
    )(_sc_body)
    out = fn(xf, s8)
    return jnp.transpose(out, (0, 2, 1))


# speedup vs baseline: 1.1014x; 1.1014x over previous
import functools

import jax
import jax.numpy as jnp
from jax import lax
from jax.experimental import pallas as pl
from jax.experimental.pallas import tpu as pltpu
from jax.experimental.pallas import tpu_sc as plsc

_K = 8
_L = 16
_NC, _NS = 2, 16
_NW = _NC * _NS
_B, _S, _C = 64, 8192, 128
_SEG8 = _S // 8
_NM64 = _S // 64
_CAP1 = 48
_CAP2 = 16
_CAP3 = 16
_NEG = float("-inf")

_SORT8 = [
    (0, 1), (2, 3), (4, 5), (6, 7),
    (0, 2), (1, 3), (4, 6), (5, 7),
    (1, 2), (5, 6),
    (0, 4), (1, 5), (2, 6), (3, 7),
    (2, 4), (3, 5),
    (1, 2), (3, 4), (5, 6),
]


def _sort8_desc(v):
    v = list(v)
    for i, j in _SORT8:
        hi = jnp.maximum(v[i], v[j])
        lo = jnp.minimum(v[i], v[j])
        v[i], v[j] = hi, lo
    return v


def _insert(r8, x):
    out = []
    carry = x
    for k in range(_K):
        hi = jnp.maximum(r8[k], carry)
        carry = jnp.minimum(r8[k], carry)
        out.append(hi)
    return out


def _eighth_of_16(vals16):
    a = _sort8_desc(vals16[:8])
    b = _sort8_desc(vals16[8:])
    top8 = [jnp.maximum(a[k], b[7 - k]) for k in range(_K)]
    t = top8[0]
    for k in range(1, _K):
        t = jnp.minimum(t, top8[k])
    return t


def _seg8_body(x_ref, o_ref):
    x = x_ref[0]
    m = x.reshape(_SEG8, 8, _C)
    o_ref[0] = jnp.max(m, axis=1)


def _seg8_max(inputs, b0, nb):
    return pl.pallas_call(
        _seg8_body,
        grid=(nb,),
        in_specs=[pl.BlockSpec((1, _S, _C), lambda b: (b + b0, 0, 0))],
        out_specs=pl.BlockSpec((1, _SEG8, _C), lambda b: (b, 0, 0)),
        out_shape=jax.ShapeDtypeStruct((nb, _SEG8, _C), jnp.float32),
    )(inputs)


def _sc_body(b0, nb, xf_hbm, s8_hbm, out_hbm,
             s8v, m64v, idb, vlb, id2b, id3b, ixb, gbuf, outb, sem):
    ntask = nb * 8 // _NW
    wid = lax.axis_index("s") * _NC + lax.axis_index("c")
    iota = lax.broadcasted_iota(jnp.int32, (_L,), 0)
    neg = jnp.full((_L,), _NEG, jnp.float32)
    zero = jnp.zeros((_L,), jnp.int32)

    def task_body(i, _):
        tau = wid * ntask + i
        b = tau // 8
        c0 = (tau % 8) * _L
        pltpu.sync_copy(s8_hbm.at[b, :, pl.ds(c0, _L)], s8v)

        def m64_body(j, accs):
            accs = list(accs)
            for u in range(16):
                jj = j * 16 + u
                m = jnp.maximum(
                    jnp.maximum(jnp.maximum(s8v[jj * 8], s8v[jj * 8 + 1]),
                                jnp.maximum(s8v[jj * 8 + 2], s8v[jj * 8 + 3])),
                    jnp.maximum(jnp.maximum(s8v[jj * 8 + 4], s8v[jj * 8 + 5]),
                                jnp.maximum(s8v[jj * 8 + 6], s8v[jj * 8 + 7])))
                m64v[jj] = m
                accs[u] = jnp.maximum(accs[u], m)
            return tuple(accs)

        accs = lax.fori_loop(0, _NM64 // 16, m64_body, (neg,) * 16)
        t1 = _eighth_of_16(list(accs))

        def c1_body(j, cnt):
            v = m64v[j]
            m = v >= t1
            ok = jnp.logical_and(m, cnt < _CAP1)
            idx = jnp.minimum(cnt, _CAP1 - 1) * _L + iota
            plsc.store_scatter(vlb, [idx], v, mask=ok)
            plsc.store_scatter(idb, [idx], zero + j, mask=ok)
            return cnt + m.astype(jnp.int32)

        cnt1 = lax.fori_loop(0, _NM64, c1_body, zero)

        def t64_body(s, r):
            v = vlb[pl.ds(pl.multiple_of(s * _L, _L), _L)]
            v = jnp.where(cnt1 > s, v, _NEG)
            return tuple(_insert(list(r), v))

        r = lax.fori_loop(0, _CAP1, t64_body, ((neg,) * _K))
        t64 = r[_K - 1]

        def c2_body(s, cnt):
            v = vlb[pl.ds(s * _L, _L)]
            sid = idb[pl.ds(s * _L, _L)]
            m = jnp.logical_and(v >= t64, cnt1 > s)
            ok = jnp.logical_and(m, cnt < _CAP2)
            idx = jnp.minimum(cnt, _CAP2 - 1) * _L + iota
            plsc.store_scatter(id2b, [idx], sid, mask=ok)
            return cnt + m.astype(jnp.int32)

        cnt2 = lax.fori_loop(0, _CAP1, c2_body, zero)

        def t8_body(s2, r):
            valid = cnt2 > s2
            sid = id2b[pl.ds(pl.multiple_of(s2 * _L, _L), _L)]
            row2 = jnp.where(valid, sid, 0) * 8
            r = list(r)
            for u in range(8):
                val = plsc.load_gather(s8v, [row2 + u, iota])
                r = _insert(r, jnp.where(valid, val, _NEG))
            return tuple(r)

        r = lax.fori_loop(0, _CAP2, t8_body, ((neg,) * _K))
        t8 = r[_K - 1]

        def c3_body(s2, cnt):
            valid = cnt2 > s2
            sid = id2b[pl.ds(pl.multiple_of(s2 * _L, _L), _L)]
            sid = jnp.where(valid, sid, 0)
            row2 = sid * 8
            for u in range(8):
                val = plsc.load_gather(s8v, [row2 + u, iota])
                m = jnp.logical_and(val >= t8, valid)
                ok = jnp.logical_and(m, cnt < _CAP3)
                idx = jnp.minimum(cnt, _CAP3 - 1) * _L + iota
                plsc.store_scatter(id3b, [idx], sid * 8 + u, mask=ok)
                cnt = cnt + m.astype(jnp.int32)
            return cnt

        cnt3 = lax.fori_loop(0, _CAP2, c3_body, zero)

        bvec = (b + b0) * (_S * _C) + c0 + iota
        for s3 in range(_CAP3):
            sid3 = id3b[pl.ds(s3 * _L, _L)]
            sid3 = jnp.where(cnt3 > s3, sid3, 0)
            base3 = sid3 * (8 * _C) + bvec
            for u in range(8):
                ixb[s3, pl.ds(u * _L, _L)] = base3 + u * _C

        copies = [pltpu.async_copy(xf_hbm.at[ixb.at[s3]], gbuf.at[s3], sem)
                  for s3 in range(_CAP3)]
        for cp in copies:
            cp.wait()

        def fin_body(s3, r):
            valid = cnt3 > s3
            r = list(r)
            for u in range(8):
                v = gbuf[s3, pl.ds(u * _L, _L)]
                r = _insert(r, jnp.where(valid, v, _NEG))
            return tuple(r)

        r8 = list(lax.fori_loop(0, _CAP3, fin_body, ((neg,) * _K)))

        for k in range(_K):
            outb[k] = r8[k]
        pltpu.sync_copy(outb, out_hbm.at[b, :, pl.ds(c0, _L)])
        return 0

    lax.fori_loop(0, ntask, task_body, 0)


def _make_sc(b0, nb):
    mesh = plsc.VectorSubcoreMesh(
        core_axis_name="c", subcore_axis_name="s",
        num_cores=_NC, num_subcores=_NS)
    return functools.partial(
        pl.kernel,
        out_type=jax.ShapeDtypeStruct((nb, _K, _C), jnp.float32),
        mesh=mesh,
        scratch_types=[
            pltpu.VMEM((_SEG8, _L), jnp.float32),
            pltpu.VMEM((_NM64, _L), jnp.float32),
            pltpu.VMEM((_CAP1 * _L,), jnp.int32),
            pltpu.VMEM((_CAP1 * _L,), jnp.float32),
            pltpu.VMEM((_CAP2 * _L,), jnp.int32),
            pltpu.VMEM((_CAP3 * _L,), jnp.int32),
            pltpu.VMEM((_CAP3, 8 * _L), jnp.int32),
            pltpu.VMEM((_CAP3, 8 * _L), jnp.float32),
            pltpu.VMEM((_K, _L), jnp.float32),
            pltpu.SemaphoreType.DMA,
        ],
        compiler_params=pltpu.CompilerParams(
            use_tc_tiling_on_sc=False, needs_layout_passes=False),
    )(functools.partial(_sc_body, b0, nb))


def kernel(inputs):
    xf = inputs.reshape(-1)
    h = _B // 2
    s8a = _seg8_max(inputs, 0, h)
    s8b = _seg8_max(inputs, h, h)
    outa = _make_sc(0, h)(xf, s8a)
    outb = _make_sc(h, h)(xf, s8b)
    out = jnp.concatenate([outa, outb], axis=0)
    return jnp.transpose(out, (0, 2, 1))
